# baseline (speedup 1.0000x reference)
import functools

import jax
import jax.numpy as jnp
from jax import lax
from jax.experimental import pallas as pl
from jax.experimental.pallas import tpu as pltpu
from jax.experimental.pallas import tpu_sc as plsc

_NC = 2
_NS = 16
_NW = _NC * _NS


def _addr_matmul(conn, bits_t, n_blk):
    n_total, k_bits = conn.shape
    in_bits, batch = bits_t.shape
    k_lo = k_bits // 2

    def kern(conn_ref, bits_ref, out_ref):
        conn_b = conn_ref[...]
        iota = lax.broadcasted_iota(jnp.int32, (n_blk, in_bits), 1)
        w_hi = jnp.zeros((n_blk, in_bits), jnp.float32)
        w_lo = jnp.zeros((n_blk, in_bits), jnp.float32)
        for k in range(k_bits):
            shift = k_bits - 1 - k
            onehot = (conn_b[:, k:k + 1] == iota)
            if shift >= k_lo:
                w_hi = w_hi + jnp.where(onehot, float(1 << (shift - k_lo)), 0.0)
            else:
                w_lo = w_lo + jnp.where(onehot, float(1 << shift), 0.0)
        bits_b = bits_ref[...].astype(jnp.bfloat16)
        hi = jnp.dot(w_hi.astype(jnp.bfloat16), bits_b,
                     preferred_element_type=jnp.float32)
        lo = jnp.dot(w_lo.astype(jnp.bfloat16), bits_b,
                     preferred_element_type=jnp.float32)
        out_ref[...] = (hi * float(1 << k_lo) + lo).astype(jnp.int32)

    return pl.pallas_call(
        kern,
        grid=(n_total // n_blk,),
        in_specs=[
            pl.BlockSpec((n_blk, k_bits), lambda i: (i, 0)),
            pl.BlockSpec((in_bits, batch), lambda i: (0, 0)),
        ],
        out_specs=pl.BlockSpec((n_blk, batch), lambda i: (i, 0)),
        out_shape=jax.ShapeDtypeStruct((n_total, batch), jnp.int32),
    )(conn, bits_t)


def _ram_gather(mem, addr_t, n_total, mem_sz, batch, threshold):
    npw = n_total // _NW
    chunk = 16
    half = chunk // 2
    rounds = npw // chunk
    mesh = plsc.VectorSubcoreMesh(core_axis_name="c", subcore_axis_name="s",
                                  num_cores=_NC, num_subcores=_NS)

    @functools.partial(
        pl.kernel,
        out_type=jax.ShapeDtypeStruct((n_total, batch), jnp.float32),
        mesh=mesh,
        scratch_types=[
            pltpu.VMEM((chunk * mem_sz,), jnp.float32),
            pltpu.VMEM((chunk, batch), jnp.int32),
            pltpu.VMEM((half, batch), jnp.float32),
        ],
        compiler_params=pltpu.CompilerParams(needs_layout_passes=False),
    )
    def k(mem_hbm, addr_hbm, out_hbm, mem_v, addr_v, out_v):
        wid = lax.axis_index("s") * _NC + lax.axis_index("c")
        base = wid * npw

        def round_body(t, carry):
            n0 = base + t * chunk
            pltpu.sync_copy(mem_hbm.at[pl.ds(n0 * mem_sz, chunk * mem_sz)],
                            mem_v)
            pltpu.sync_copy(addr_hbm.at[pl.ds(n0, chunk)], addr_v)
            for h in range(2):
                for c in range(half):
                    cc = h * half + c
                    row = mem_v.at[pl.ds(cc * mem_sz, mem_sz)]

                    def jbody(j, carry2, c=c, cc=cc, row=row):
                        off = pl.multiple_of(j * 16, 16)
                        idx = addr_v[cc, pl.ds(off, 16)]
                        vals = plsc.load_gather(row, [idx])
                        if threshold:
                            vals = jnp.where(vals > 0.5, 1.0, 0.0)
                        out_v[c, pl.ds(off, 16)] = vals
                        return carry2

                    lax.fori_loop(0, batch // 16, jbody, 0)
                pltpu.sync_copy(out_v, out_hbm.at[pl.ds(n0 + h * half, half)])
            return carry

        lax.fori_loop(0, rounds, round_body, 0)

    return k(mem.reshape(-1), addr_t)


def _ram_gather_out(mem, addr_t, n_out, mem_sz, batch):
    mesh = plsc.VectorSubcoreMesh(core_axis_name="c", subcore_axis_name="s",
                                  num_cores=_NC, num_subcores=_NS)

    @functools.partial(
        pl.kernel,
        out_type=jax.ShapeDtypeStruct((n_out, batch), jnp.float32),
        mesh=mesh,
        scratch_types=[
            pltpu.VMEM((mem_sz,), jnp.float32),
            pltpu.VMEM((1, batch), jnp.int32),
            pltpu.VMEM((1, batch), jnp.float32),
        ],
        compiler_params=pltpu.CompilerParams(needs_layout_passes=False),
    )
    def k(mem_hbm, addr_hbm, out_hbm, mem_v, addr_v, out_v):
        wid = lax.axis_index("s") * _NC + lax.axis_index("c")

        @pl.when(wid < n_out)
        def _():
            pltpu.sync_copy(mem_hbm.at[pl.ds(wid * mem_sz, mem_sz)], mem_v)
            pltpu.sync_copy(addr_hbm.at[pl.ds(wid, 1)], addr_v)

            def jbody(j, carry):
                off = pl.multiple_of(j * 16, 16)
                idx = addr_v[0, pl.ds(off, 16)]
                out_v[0, pl.ds(off, 16)] = plsc.load_gather(mem_v, [idx])
                return carry

            lax.fori_loop(0, batch // 16, jbody, 0)
            pltpu.sync_copy(out_v, out_hbm.at[pl.ds(wid, 1)])

    return k(mem.reshape(-1), addr_t)


def kernel(x, conn0, conn1, conn2, mem0, mem1, mem2):
    batch = x.shape[0]
    hidden, mem_sz = mem0.shape
    n_out = mem2.shape[0]
    x_t = x.T.astype(jnp.float32)
    addr0 = _addr_matmul(conn0, x_t, n_blk=256)
    b0_t = _ram_gather(mem0, addr0, hidden, mem_sz, batch, threshold=True)
    addr1 = _addr_matmul(conn1, b0_t, n_blk=256)
    b1_t = _ram_gather(mem1, addr1, hidden, mem_sz, batch, threshold=True)
    conn2p = jnp.pad(conn2, ((0, 32 - n_out), (0, 0)))
    addr2 = _addr_matmul(conn2p, b1_t, n_blk=32)
    v2_t = _ram_gather_out(mem2, addr2, n_out, mem_sz, batch)
    return v2_t.T


# speedup vs baseline: 1.0900x; 1.0900x over previous
import functools

import jax
import jax.numpy as jnp
from jax import lax
from jax.experimental import pallas as pl
from jax.experimental.pallas import tpu as pltpu
from jax.experimental.pallas import tpu_sc as plsc

_NC = 2
_NS = 16
_NW = _NC * _NS


def _addr_matmul(conn, bits_t, n_blk):
    n_total, k_bits = conn.shape
    in_bits, batch = bits_t.shape
    k_lo = k_bits // 2

    def kern(conn_ref, bits_ref, out_ref):
        conn_b = conn_ref[...]
        iota = lax.broadcasted_iota(jnp.int32, (n_blk, in_bits), 1)
        w_hi = jnp.zeros((n_blk, in_bits), jnp.float32)
        w_lo = jnp.zeros((n_blk, in_bits), jnp.float32)
        for k in range(k_bits):
            shift = k_bits - 1 - k
            onehot = (conn_b[:, k:k + 1] == iota)
            if shift >= k_lo:
                w_hi = w_hi + jnp.where(onehot, float(1 << (shift - k_lo)), 0.0)
            else:
                w_lo = w_lo + jnp.where(onehot, float(1 << shift), 0.0)
        bits_b = bits_ref[...].astype(jnp.bfloat16)
        hi = jnp.dot(w_hi.astype(jnp.bfloat16), bits_b,
                     preferred_element_type=jnp.float32)
        lo = jnp.dot(w_lo.astype(jnp.bfloat16), bits_b,
                     preferred_element_type=jnp.float32)
        out_ref[...] = (hi * float(1 << k_lo) + lo).astype(jnp.int32)

    return pl.pallas_call(
        kern,
        grid=(n_total // n_blk,),
        in_specs=[
            pl.BlockSpec((n_blk, k_bits), lambda i: (i, 0)),
            pl.BlockSpec((in_bits, batch), lambda i: (0, 0)),
        ],
        out_specs=pl.BlockSpec((n_blk, batch), lambda i: (i, 0)),
        out_shape=jax.ShapeDtypeStruct((n_total, batch), jnp.int32),
    )(conn, bits_t)


def _ram_gather(mem, addr_t, n_total, mem_sz, batch, threshold):
    npw = n_total // _NW
    chunk = 16
    half = chunk // 2
    rounds = npw // chunk
    mesh = plsc.VectorSubcoreMesh(core_axis_name="c", subcore_axis_name="s",
                                  num_cores=_NC, num_subcores=_NS)

    @functools.partial(
        pl.kernel,
        out_type=jax.ShapeDtypeStruct((n_total, batch), jnp.float32),
        mesh=mesh,
        scratch_types=[
            pltpu.VMEM((chunk, mem_sz), jnp.float32),
            pltpu.VMEM((chunk, batch), jnp.int32),
            pltpu.VMEM((half, batch), jnp.float32),
        ],
        compiler_params=pltpu.CompilerParams(needs_layout_passes=False),
    )
    def k(mem_hbm, addr_hbm, out_hbm, mem_v, addr_v, out_v):
        wid = lax.axis_index("s") * _NC + lax.axis_index("c")
        base = wid * npw

        def round_body(t, carry):
            n0 = base + t * chunk
            pltpu.sync_copy(mem_hbm.at[pl.ds(n0, chunk)], mem_v)
            pltpu.sync_copy(addr_hbm.at[pl.ds(n0, chunk)], addr_v)
            for h in range(2):
                for c in range(half):
                    cc = h * half + c
                    cvec = jnp.full((16,), cc, jnp.int32)

                    def jbody(j, carry2, c=c, cvec=cvec, cc=cc):
                        off = pl.multiple_of(j * 16, 16)
                        idx = addr_v[cc, pl.ds(off, 16)]
                        vals = plsc.load_gather(mem_v, [cvec, idx])
                        if threshold:
                            vals = jnp.where(vals > 0.5, 1.0, 0.0)
                        out_v[c, pl.ds(off, 16)] = vals
                        return carry2

                    lax.fori_loop(0, batch // 16, jbody, 0)
                pltpu.sync_copy(out_v, out_hbm.at[pl.ds(n0 + h * half, half)])
            return carry

        lax.fori_loop(0, rounds, round_body, 0)

    return k(mem, addr_t)


def _ram_gather_out(mem, addr_t, n_out, mem_sz, batch):
    mesh = plsc.VectorSubcoreMesh(core_axis_name="c", subcore_axis_name="s",
                                  num_cores=_NC, num_subcores=_NS)

    @functools.partial(
        pl.kernel,
        out_type=jax.ShapeDtypeStruct((n_out, batch), jnp.float32),
        mesh=mesh,
        scratch_types=[
            pltpu.VMEM((1, mem_sz), jnp.float32),
            pltpu.VMEM((1, batch), jnp.int32),
            pltpu.VMEM((1, batch), jnp.float32),
        ],
        compiler_params=pltpu.CompilerParams(needs_layout_passes=False),
    )
    def k(mem_hbm, addr_hbm, out_hbm, mem_v, addr_v, out_v):
        wid = lax.axis_index("s") * _NC + lax.axis_index("c")

        @pl.when(wid < n_out)
        def _():
            pltpu.sync_copy(mem_hbm.at[pl.ds(wid, 1)], mem_v)
            pltpu.sync_copy(addr_hbm.at[pl.ds(wid, 1)], addr_v)
            zvec = jnp.zeros((16,), jnp.int32)

            def jbody(j, carry):
                off = pl.multiple_of(j * 16, 16)
                idx = addr_v[0, pl.ds(off, 16)]
                out_v[0, pl.ds(off, 16)] = plsc.load_gather(mem_v, [zvec, idx])
                return carry

            lax.fori_loop(0, batch // 16, jbody, 0)
            pltpu.sync_copy(out_v, out_hbm.at[pl.ds(wid, 1)])

    return k(mem, addr_t)


def kernel(x, conn0, conn1, conn2, mem0, mem1, mem2):
    batch = x.shape[0]
    hidden, mem_sz = mem0.shape
    n_out = mem2.shape[0]
    x_t = x.T.astype(jnp.float32)
    addr0 = _addr_matmul(conn0, x_t, n_blk=256)
    b0_t = _ram_gather(mem0, addr0, hidden, mem_sz, batch, threshold=True)
    addr1 = _addr_matmul(conn1, b0_t, n_blk=256)
    b1_t = _ram_gather(mem1, addr1, hidden, mem_sz, batch, threshold=True)
    conn2p = jnp.pad(conn2, ((0, 32 - n_out), (0, 0)))
    addr2 = _addr_matmul(conn2p, b1_t, n_blk=32)
    v2_t = _ram_gather_out(mem2, addr2, n_out, mem_sz, batch)
    return v2_t.T


# speedup vs baseline: 1.0993x; 1.0085x over previous
import functools

import jax
import jax.numpy as jnp
from jax import lax
from jax.experimental import pallas as pl
from jax.experimental.pallas import tpu as pltpu
from jax.experimental.pallas import tpu_sc as plsc

_NC = 2
_NS = 16
_NW = _NC * _NS


def _addr_matmul(conn, bits_t, n_blk):
    n_total, k_bits = conn.shape
    in_bits, batch = bits_t.shape
    k_lo = k_bits // 2

    def kern(conn_ref, bits_ref, out_ref):
        conn_b = conn_ref[...]
        iota = lax.broadcasted_iota(jnp.int32, (n_blk, in_bits), 1)
        w_hi = jnp.zeros((n_blk, in_bits), jnp.float32)
        w_lo = jnp.zeros((n_blk, in_bits), jnp.float32)
        for k in range(k_bits):
            shift = k_bits - 1 - k
            onehot = (conn_b[:, k:k + 1] == iota)
            if shift >= k_lo:
                w_hi = w_hi + jnp.where(onehot, float(1 << (shift - k_lo)), 0.0)
            else:
                w_lo = w_lo + jnp.where(onehot, float(1 << shift), 0.0)
        bits_b = bits_ref[...].astype(jnp.bfloat16)
        hi = jnp.dot(w_hi.astype(jnp.bfloat16), bits_b,
                     preferred_element_type=jnp.float32)
        lo = jnp.dot(w_lo.astype(jnp.bfloat16), bits_b,
                     preferred_element_type=jnp.float32)
        out_ref[...] = (hi * float(1 << k_lo) + lo).astype(jnp.int32)

    return pl.pallas_call(
        kern,
        grid=(n_total // n_blk,),
        in_specs=[
            pl.BlockSpec((n_blk, k_bits), lambda i: (i, 0)),
            pl.BlockSpec((in_bits, batch), lambda i: (0, 0)),
        ],
        out_specs=pl.BlockSpec((n_blk, batch), lambda i: (i, 0)),
        out_shape=jax.ShapeDtypeStruct((n_total, batch), jnp.int32),
    )(conn, bits_t)


def _ram_gather(mem, addr_t, n_total, mem_sz, batch, threshold):
    npw = n_total // _NW
    chunk = 16
    half = chunk // 2
    rounds = npw // chunk
    mesh = plsc.VectorSubcoreMesh(core_axis_name="c", subcore_axis_name="s",
                                  num_cores=_NC, num_subcores=_NS)

    @functools.partial(
        pl.kernel,
        out_type=jax.ShapeDtypeStruct((n_total, batch), jnp.float32),
        mesh=mesh,
        scratch_types=[
            pltpu.VMEM((chunk, mem_sz), jnp.float32),
            pltpu.VMEM((chunk, batch), jnp.int32),
            pltpu.VMEM((half, batch), jnp.float32),
        ],
        compiler_params=pltpu.CompilerParams(needs_layout_passes=False),
    )
    def k(mem_hbm, addr_hbm, out_hbm, mem_v, addr_v, out_v):
        wid = lax.axis_index("s") * _NC + lax.axis_index("c")
        base = wid * npw

        def round_body(t, carry):
            n0 = base + t * chunk
            pltpu.sync_copy(mem_hbm.at[pl.ds(n0, chunk)], mem_v)
            pltpu.sync_copy(addr_hbm.at[pl.ds(n0, chunk)], addr_v)
            for h in range(2):
                for c in range(half):
                    cc = h * half + c
                    cvec = jnp.full((16,), cc, jnp.int32)

                    def jbody(j, carry2, c=c, cvec=cvec, cc=cc):
                        off = pl.multiple_of(j * 16, 16)
                        idx = addr_v[cc, pl.ds(off, 16)]
                        vals = plsc.load_gather(mem_v, [cvec, idx])
                        if threshold:
                            vals = jnp.where(vals > 0.5, 1.0, 0.0)
                        out_v[c, pl.ds(off, 16)] = vals
                        return carry2

                    lax.fori_loop(0, batch // 16, jbody, 0)
                pltpu.sync_copy(out_v, out_hbm.at[pl.ds(n0 + h * half, half)])
            return carry

        lax.fori_loop(0, rounds, round_body, 0)

    return k(mem, addr_t)


def _ram_gather_out(mem, addr_t, n_out, mem_sz, batch):
    mesh = plsc.VectorSubcoreMesh(core_axis_name="c", subcore_axis_name="s",
                                  num_cores=_NC, num_subcores=_NS)

    @functools.partial(
        pl.kernel,
        out_type=jax.ShapeDtypeStruct((n_out, batch), jnp.float32),
        mesh=mesh,
        scratch_types=[
            pltpu.VMEM((1, mem_sz), jnp.float32),
            pltpu.VMEM((1, batch), jnp.int32),
            pltpu.VMEM((1, batch), jnp.float32),
        ],
        compiler_params=pltpu.CompilerParams(needs_layout_passes=False),
    )
    def k(mem_hbm, addr_hbm, out_hbm, mem_v, addr_v, out_v):
        wid = lax.axis_index("s") * _NC + lax.axis_index("c")

        @pl.when(wid < n_out)
        def _():
            pltpu.sync_copy(mem_hbm.at[pl.ds(wid, 1)], mem_v)
            pltpu.sync_copy(addr_hbm.at[pl.ds(wid, 1)], addr_v)
            zvec = jnp.zeros((16,), jnp.int32)

            def jbody(j, carry):
                off = pl.multiple_of(j * 16, 16)
                idx = addr_v[0, pl.ds(off, 16)]
                out_v[0, pl.ds(off, 16)] = plsc.load_gather(mem_v, [zvec, idx])
                return carry

            lax.fori_loop(0, batch // 16, jbody, 0)
            pltpu.sync_copy(out_v, out_hbm.at[pl.ds(wid, 1)])

    return k(mem, addr_t)


def kernel(x, conn0, conn1, conn2, mem0, mem1, mem2):
    batch = x.shape[0]
    hidden, mem_sz = mem0.shape
    n_out = mem2.shape[0]
    x_t = x.T.astype(jnp.float32)
    addr0 = _addr_matmul(conn0, x_t, n_blk=512)
    b0_t = _ram_gather(mem0, addr0, hidden, mem_sz, batch, threshold=True)
    addr1 = _addr_matmul(conn1, b0_t, n_blk=512)
    b1_t = _ram_gather(mem1, addr1, hidden, mem_sz, batch, threshold=True)
    conn2p = jnp.pad(conn2, ((0, 32 - n_out), (0, 0)))
    addr2 = _addr_matmul(conn2p, b1_t, n_blk=32)
    v2_t = _ram_gather_out(mem2, addr2, n_out, mem_sz, batch)
    return v2_t.T
